# baseline (speedup 1.0000x reference)
import jax
import jax.numpy as jnp
from jax import lax
from jax.experimental import pallas as pl
from jax.experimental.pallas import tpu as pltpu
from jax.experimental.pallas import tpu_sc as plsc

EMB = 64
SCALE = 8.0
VOCAB = 1_000_000
NW = 32
NTILE = VOCAB // 128
TPW = (NTILE + NW - 1) // NW
VPAD = NTILE * 128 + 128

NI = 16384
ICHUNK = 128
NJ = 50


def _tp_body(tt_hbm, tail_hbm, tlin_hbm, blk_v, obuf_v, rsem, osem):
    c = lax.axis_index("c")
    s = lax.axis_index("s")
    wid = s * 2 + c

    iota = lax.iota(jnp.int32, 16)
    vrows = [vg * 16 + iota for vg in range(8)]

    def bnum(t):
        return wid + t * NW

    def fire_read(t, p):
        @pl.when(bnum(t) < NTILE)
        def _():
            pltpu.make_async_copy(
                tt_hbm.at[:, pl.ds(bnum(t) * 128, 128)],
                blk_v.at[p],
                rsem.at[p],
            ).start()

    def wait_read(t, p):
        @pl.when(bnum(t) < NTILE)
        def _():
            pltpu.make_async_copy(
                tt_hbm.at[:, pl.ds(0, 128)], blk_v.at[p], rsem.at[p]
            ).wait()

    def out_desc(t, p):
        return pltpu.make_async_copy(
            obuf_v.at[p, :, pl.ds(0, 128)],
            tlin_hbm.at[pl.ds(bnum(t) * 128, 128)],
            osem.at[p],
        )

    def transpose_block(p):
        @plsc.parallel_loop(0, 64, unroll=4)
        def _(k):
            colk = jnp.full((16,), k, dtype=jnp.int32)
            for vg in range(8):
                vals = blk_v[p, k, pl.ds(vg * 16, 16)]
                plsc.store_scatter(
                    obuf_v.at[p], [vrows[vg], colk], vals * SCALE
                )

    def step(t, p):
        wait_read(t, p)

        @pl.when(bnum(t) < NTILE)
        def _():
            @pl.when(t >= 2)
            def _():
                out_desc(t, p).wait()

            transpose_block(p)
            out_desc(t, p).start()

        fire_read(t + 2, p)

    fire_read(0, 0)
    fire_read(1, 1)

    def blk2(i, carry):
        step(2 * i, 0)
        step(2 * i + 1, 1)
        return carry

    lax.fori_loop(0, TPW // 2, blk2, 0)
    step(TPW - 1, 0)

    @pl.when(bnum(TPW - 1) < NTILE)
    def _():
        out_desc(0, 0).wait()

    @pl.when(bnum(TPW - 2) < NTILE)
    def _():
        out_desc(0, 1).wait()

    @pl.when(wid == 0)
    def _():
        pltpu.sync_copy(tail_hbm, blk_v.at[0])
        transpose_block(0)
        pltpu.sync_copy(
            obuf_v.at[0, pl.ds(0, 64), pl.ds(0, 128)],
            tlin_hbm.at[pl.ds(NTILE * 128, 64)],
        )


def _gather_body(xl_hbm, tlin_hbm, out_hbm, idx_v, grows_v, brick_v, gsem, osem):
    c = lax.axis_index("c")
    s = lax.axis_index("s")
    wid = s * 2 + c

    iota = lax.iota(jnp.int32, 16)
    krows = [kg * 16 + iota for kg in range(4)]

    pltpu.sync_copy(xl_hbm.at[:, pl.ds(4 * wid, 4)], idx_v)

    def fire_gather(j, h):
        pltpu.make_async_copy(
            tlin_hbm.at[idx_v.at[j, h]],
            grows_v.at[h % 2],
            gsem.at[h % 2],
        ).start()

    def wait_gather(h):
        pltpu.make_async_copy(
            tlin_hbm.at[idx_v.at[0, 0]],
            grows_v.at[h % 2],
            gsem.at[h % 2],
        ).wait()

    def brick_desc(j, h):
        return pltpu.make_async_copy(
            brick_v.at[h % 2, :, pl.ds(0, ICHUNK)],
            out_hbm.at[j, :, pl.ds(wid * 512 + h * ICHUNK, ICHUNK)],
            osem.at[h % 2],
        )

    def transpose_unit(h):
        @plsc.parallel_loop(0, ICHUNK, unroll=4)
        def _(ip):
            coli = jnp.full((16,), ip, dtype=jnp.int32)
            for kg in range(4):
                vals = grows_v[h % 2, ip, pl.ds(kg * 16, 16)]
                plsc.store_scatter(
                    brick_v.at[h % 2], [krows[kg], coli], vals
                )

    def unit(j, h, jn, hn, first, last):
        wait_gather(h)
        if not last:
            fire_gather(jn, hn)
        if not first:
            brick_desc(0, h).wait()
        transpose_unit(h)
        brick_desc(j, h).start()

    fire_gather(0, 0)

    def j_loop(j, carry):
        @pl.when(j == 0)
        def _():
            unit(0, 0, 0, 1, True, False)
            unit(0, 1, 0, 2, True, False)
            unit(0, 2, 0, 3, False, False)
            unit(0, 3, 1, 0, False, False)

        @pl.when(j > 0)
        def _():
            unit(j, 0, j, 1, False, False)
            unit(j, 1, j, 2, False, False)
            unit(j, 2, j, 3, False, False)
            unit(j, 3, j + 1, 0, False, False)

        return carry

    lax.fori_loop(0, NJ - 1, j_loop, 0)
    unit(NJ - 1, 0, NJ - 1, 1, False, False)
    unit(NJ - 1, 1, NJ - 1, 2, False, False)
    unit(NJ - 1, 2, NJ - 1, 3, False, False)
    unit(NJ - 1, 3, 0, 0, False, True)

    brick_desc(0, 0).wait()
    brick_desc(0, 1).wait()


def kernel(x, table):
    mesh = plsc.VectorSubcoreMesh(core_axis_name="c", subcore_axis_name="s")

    tt = table.T
    tail = jnp.pad(tt[:, NTILE * 128:], ((0, 0), (0, 64)))

    tlin = pl.kernel(
        _tp_body,
        out_type=jax.ShapeDtypeStruct((VPAD, 128), jnp.float32),
        mesh=mesh,
        compiler_params=pltpu.CompilerParams(
            use_tc_tiling_on_sc=True, needs_layout_passes=False
        ),
        scratch_types=[
            pltpu.VMEM((2, 64, 128), jnp.float32),
            pltpu.VMEM((2, 128, 129), jnp.float32),
            pltpu.SemaphoreType.DMA((2,)),
            pltpu.SemaphoreType.DMA((2,)),
        ],
    )(tt, tail)

    xl = x.T.astype(jnp.int32).reshape(NJ, NI // 128, 128)

    ot = pl.kernel(
        _gather_body,
        out_type=jax.ShapeDtypeStruct((NJ, EMB, NI), jnp.float32),
        mesh=mesh,
        compiler_params=pltpu.CompilerParams(
            use_tc_tiling_on_sc=True, needs_layout_passes=False
        ),
        scratch_types=[
            pltpu.VMEM((NJ, 4, 128), jnp.int32),
            pltpu.VMEM((2, ICHUNK, 128), jnp.float32),
            pltpu.VMEM((2, EMB, ICHUNK + 1), jnp.float32),
            pltpu.SemaphoreType.DMA((2,)),
            pltpu.SemaphoreType.DMA((2,)),
        ],
    )(xl, tlin)

    return jnp.transpose(ot, (2, 0, 1))


# speedup vs baseline: 1.0421x; 1.0421x over previous
import jax
import jax.numpy as jnp
from jax import lax
from jax.experimental import pallas as pl
from jax.experimental.pallas import tpu as pltpu
from jax.experimental.pallas import tpu_sc as plsc

EMB = 64
SCALE = 8.0
VOCAB = 1_000_000

NW = 32
GSZ = 128
NGRP = 200
IDX_PER_W = NGRP * GSZ
TOTAL = NW * IDX_PER_W
ORPG = GSZ * EMB // 128


def _emb_body(xr_hbm, tpad_hbm, out_hbm, idx_v, grows_v, orows_v, gsem, osem):
    c = lax.axis_index("c")
    s = lax.axis_index("s")
    wid = s * 2 + c
    obase = wid * (IDX_PER_W * EMB // 128)

    pltpu.sync_copy(xr_hbm.at[wid], idx_v)

    def fire_gather(g, p):
        pltpu.make_async_copy(
            tpad_hbm.at[idx_v.at[g]], grows_v.at[p], gsem.at[p]
        ).start()

    def wait_gather(p):
        pltpu.make_async_copy(
            tpad_hbm.at[idx_v.at[0]], grows_v.at[p], gsem.at[p]
        ).wait()

    def scale_slot(p):
        def body(r2, carry):
            for cc in range(4):
                sl = pl.ds(cc * 16, 16)
                sh = pl.ds(64 + cc * 16, 16)
                orows_v[p, r2, sl] = grows_v[p, 2 * r2, sl] * SCALE
                orows_v[p, r2, sh] = grows_v[p, 2 * r2 + 1, sl] * SCALE
            return carry

        lax.fori_loop(0, ORPG, body, 0)

    def out_desc(g, p):
        return pltpu.make_async_copy(
            orows_v.at[p],
            out_hbm.at[pl.ds(obase + g * ORPG, ORPG)],
            osem.at[p],
        )

    def unit(g, p, fire_next, wait_out):
        wait_gather(p)
        if fire_next:
            fire_gather(g + 1, 1 - p)
        if wait_out:
            out_desc(g, p).wait()
        scale_slot(p)
        out_desc(g, p).start()

    fire_gather(0, 0)
    unit(0, 0, True, False)
    unit(1, 1, True, False)

    def main_blk(i, carry):
        g0 = 2 + 2 * i
        unit(g0, 0, True, True)
        unit(g0 + 1, 1, True, True)
        return carry

    lax.fori_loop(0, (NGRP - 4) // 2, main_blk, 0)

    unit(NGRP - 2, 0, True, True)
    unit(NGRP - 1, 1, False, True)

    out_desc(0, 0).wait()
    out_desc(0, 1).wait()


def kernel(x, table):
    xr = x.astype(jnp.int32).reshape(NW, NGRP, GSZ)
    tpad = jnp.pad(table, ((0, 0), (0, 128 - EMB)))
    mesh = plsc.VectorSubcoreMesh(core_axis_name="c", subcore_axis_name="s")
    out = pl.kernel(
        _emb_body,
        out_type=jax.ShapeDtypeStruct((TOTAL * EMB // 128, 128), jnp.float32),
        mesh=mesh,
        compiler_params=pltpu.CompilerParams(
            use_tc_tiling_on_sc=True, needs_layout_passes=False
        ),
        scratch_types=[
            pltpu.VMEM((NGRP, GSZ), jnp.int32),
            pltpu.VMEM((2, GSZ, 128), jnp.float32),
            pltpu.VMEM((2, ORPG, 128), jnp.float32),
            pltpu.SemaphoreType.DMA((2,)),
            pltpu.SemaphoreType.DMA((2,)),
        ],
    )(xr, tpad)
    return out.reshape(x.shape[0], x.shape[1], EMB)


# speedup vs baseline: 1.2658x; 1.2146x over previous
import jax
import jax.numpy as jnp
from jax import lax
from jax.experimental import pallas as pl
from jax.experimental.pallas import tpu as pltpu
from jax.experimental.pallas import tpu_sc as plsc

EMB = 64
SCALE = 8.0

NW = 32
GSZ = 128
GRP = 256
NGRP = 100
NSLOT = 4
IDX_PER_W = NGRP * GRP
TOTAL = NW * IDX_PER_W
NIDXROW = IDX_PER_W // GSZ


def _emb_body(xr_hbm, table_hbm, out_hbm, idx_v, rows_v, gsem, osem):
    c = lax.axis_index("c")
    s = lax.axis_index("s")
    wid = s * 2 + c
    base = wid * IDX_PER_W

    pltpu.sync_copy(xr_hbm.at[wid], idx_v)

    def fire_gather(g, slot):
        for h in range(2):
            pltpu.make_async_copy(
                table_hbm.at[idx_v.at[2 * g + h]],
                rows_v.at[pl.ds(slot * GRP + h * GSZ, GSZ)],
                gsem.at[slot],
            ).start()

    def wait_gather(slot):
        for h in range(2):
            pltpu.make_async_copy(
                table_hbm.at[idx_v.at[h]],
                rows_v.at[pl.ds(slot * GRP + h * GSZ, GSZ)],
                gsem.at[slot],
            ).wait()

    def scale_slot(slot):
        sb = slot * GRP

        def body(i, carry):
            r = sb + i * 4
            for rr in range(4):
                for cc in range(4):
                    sl = pl.ds(cc * 16, 16)
                    rows_v[r + rr, sl] = rows_v[r + rr, sl] * SCALE
            return carry

        lax.fori_loop(0, GRP // 4, body, 0)

    def out_desc(g, slot):
        return pltpu.make_async_copy(
            rows_v.at[pl.ds(slot * GRP, GRP)],
            out_hbm.at[pl.ds(base + g * GRP, GRP)],
            osem.at[slot],
        )

    def consume(g, slot):
        wait_gather(slot)
        scale_slot(slot)
        out_desc(g, slot).start()

    fire_gather(0, 0)
    fire_gather(1, 1)

    fire_gather(2, 2)
    consume(0, 0)
    fire_gather(3, 3)
    consume(1, 1)

    def main_blk(i, carry):
        g0 = 2 + i * 4
        for db in range(4):
            g = g0 + db
            slot = (2 + db) % 4
            fslot = db
            out_desc(g, fslot).wait()
            fire_gather(g + 2, fslot)
            consume(g, slot)
        return carry

    lax.fori_loop(0, 24, main_blk, 0)

    consume(98, 2)
    consume(99, 3)

    for slot in range(4):
        out_desc(0, slot).wait()


def kernel(x, table):
    xr = x.astype(jnp.int32).reshape(NW, NIDXROW, GSZ)
    mesh = plsc.VectorSubcoreMesh(core_axis_name="c", subcore_axis_name="s")
    out = pl.kernel(
        _emb_body,
        out_type=jax.ShapeDtypeStruct((TOTAL, EMB), jnp.float32),
        mesh=mesh,
        compiler_params=pltpu.CompilerParams(use_tc_tiling_on_sc=False),
        scratch_types=[
            pltpu.VMEM((NIDXROW, GSZ), jnp.int32),
            pltpu.VMEM((NSLOT * GRP, EMB), jnp.float32),
            pltpu.SemaphoreType.DMA((NSLOT,)),
            pltpu.SemaphoreType.DMA((NSLOT,)),
        ],
    )(xr, table)
    return out.reshape(x.shape[0], x.shape[1], EMB)
